# Initial kernel scaffold; baseline (speedup 1.0000x reference)
import jax
import jax.numpy as jnp
from jax.experimental import pallas as pl


def kernel(x, edge_weight, params, edge_index):
    raise NotImplementedError("write your pallas kernel here")


# speedup vs baseline: 159.8336x; 159.8336x over previous
import jax
import jax.numpy as jnp
from jax.experimental import pallas as pl

_PREC = jax.lax.Precision.HIGHEST


def _predictor_kernel(x_ref,
                      wi1_ref, wc1_ref, wo1_ref, bi1_ref, bc1_ref, bo1_ref, wco1_ref,
                      wi2_ref, wc2_ref, wo2_ref, bi2_ref, bc2_ref, bo2_ref, wco2_ref,
                      w1a_ref, b1_ref, w1b_ref, w2_ref, b2_ref,
                      out_ref):
    h = x_ref[:]
    for (wi, wc, wo, bi, bc, bo, wco) in (
        (wi1_ref, wc1_ref, wo1_ref, bi1_ref, bc1_ref, bo1_ref, wco1_ref),
        (wi2_ref, wc2_ref, wo2_ref, bi2_ref, bc2_ref, bo2_ref, wco2_ref),
    ):
        gi = jax.nn.sigmoid(
            jnp.dot(h, wi[:], precision=_PREC, preferred_element_type=jnp.float32)
            + bi[:])
        gt = jnp.tanh(
            jnp.dot(h, wc[:], precision=_PREC, preferred_element_type=jnp.float32)
            + bc[:])
        c = gi * gt
        go = jax.nn.sigmoid(
            jnp.dot(h, wo[:], precision=_PREC, preferred_element_type=jnp.float32)
            + wco[:] * c + bo[:])
        h = go * jnp.tanh(c)

    a = jnp.dot(h, w1a_ref[:], precision=_PREC,
                preferred_element_type=jnp.float32) + b1_ref[:]
    bt = jnp.dot(h, w1b_ref[:], precision=_PREC,
                 preferred_element_type=jnp.float32).T
    w2 = w2_ref[:]
    n = a.shape[0]
    acc = jnp.broadcast_to(b2_ref[:], (n, n))
    for k in range(a.shape[1]):
        t = jnp.maximum(a[:, k:k + 1] + bt[k:k + 1, :], 0.0)
        acc = acc + t * w2[0:1, k:k + 1]
    out_ref[:] = jax.nn.sigmoid(acc)


def kernel(x, edge_weight, params, edge_index):
    del edge_weight, edge_index
    n = x.shape[0]
    hid = params["lp_w2"].shape[0]

    args = [x]
    for p in params["layers"]:
        args += [
            p["W_i"], p["W_c"], p["W_o"],
            (p["b_i"] + p["conv_i_b"]).astype(jnp.float32),
            (p["b_c"] + p["conv_c_b"]).astype(jnp.float32),
            (p["b_o"] + p["conv_o_b"]).astype(jnp.float32),
            p["w_c_o"],
        ]
    args += [
        params["lp_w1"][:hid],
        params["lp_b1"][None, :],
        params["lp_w1"][hid:],
        params["lp_w2"].reshape(1, hid),
        params["lp_b2"].reshape(1, 1),
    ]

    return pl.pallas_call(
        _predictor_kernel,
        out_shape=jax.ShapeDtypeStruct((n, n), jnp.float32),
    )(*args)


# speedup vs baseline: 168.6391x; 1.0551x over previous
import jax
import jax.numpy as jnp
from jax.experimental import pallas as pl
from jax.experimental.pallas import tpu as pltpu

_PREC = jax.lax.Precision.HIGHEST


def _embed_kernel(x_ref,
                  wi1_ref, wc1_ref, wo1_ref, bi1_ref, bc1_ref, bo1_ref, wco1_ref,
                  wi2_ref, wc2_ref, wo2_ref, bi2_ref, bc2_ref, bo2_ref, wco2_ref,
                  w1a_ref, b1_ref, w1b_ref,
                  a_ref, bt_ref):
    h = x_ref[:]
    for (wi, wc, wo, bi, bc, bo, wco) in (
        (wi1_ref, wc1_ref, wo1_ref, bi1_ref, bc1_ref, bo1_ref, wco1_ref),
        (wi2_ref, wc2_ref, wo2_ref, bi2_ref, bc2_ref, bo2_ref, wco2_ref),
    ):
        gi = jax.nn.sigmoid(
            jnp.dot(h, wi[:], precision=_PREC, preferred_element_type=jnp.float32)
            + bi[:])
        gt = jnp.tanh(
            jnp.dot(h, wc[:], precision=_PREC, preferred_element_type=jnp.float32)
            + bc[:])
        c = gi * gt
        go = jax.nn.sigmoid(
            jnp.dot(h, wo[:], precision=_PREC, preferred_element_type=jnp.float32)
            + wco[:] * c + bo[:])
        h = go * jnp.tanh(c)

    a_ref[:] = jnp.dot(h, w1a_ref[:], precision=_PREC,
                       preferred_element_type=jnp.float32) + b1_ref[:]
    bt_ref[:] = jnp.dot(h, w1b_ref[:], precision=_PREC,
                        preferred_element_type=jnp.float32).T


def _score_kernel(a_ref, bt_ref, w2_ref, b2_ref, out_ref):
    a = a_ref[:]
    bt = bt_ref[:]
    w2 = w2_ref[:]
    acc = jnp.broadcast_to(b2_ref[:], out_ref.shape)
    for k in range(a.shape[1]):
        t = jnp.maximum(a[:, k:k + 1] + bt[k:k + 1, :], 0.0)
        acc = acc + t * w2[0:1, k:k + 1]
    out_ref[:] = jax.nn.sigmoid(acc)


def kernel(x, edge_weight, params, edge_index):
    del edge_weight, edge_index
    n = x.shape[0]
    hid = params["lp_w2"].shape[0]

    args = [x]
    for p in params["layers"]:
        args += [
            p["W_i"], p["W_c"], p["W_o"],
            (p["b_i"] + p["conv_i_b"]).astype(jnp.float32),
            (p["b_c"] + p["conv_c_b"]).astype(jnp.float32),
            (p["b_o"] + p["conv_o_b"]).astype(jnp.float32),
            p["w_c_o"],
        ]
    args += [
        params["lp_w1"][:hid],
        params["lp_b1"][None, :],
        params["lp_w1"][hid:],
    ]

    a, bt = pl.pallas_call(
        _embed_kernel,
        out_shape=(jax.ShapeDtypeStruct((n, hid), jnp.float32),
                   jax.ShapeDtypeStruct((hid, n), jnp.float32)),
    )(*args)

    blk = 128
    grid = n // blk
    return pl.pallas_call(
        _score_kernel,
        grid=(grid,),
        in_specs=[
            pl.BlockSpec((blk, hid), lambda i: (i, 0)),
            pl.BlockSpec((hid, n), lambda i: (0, 0)),
            pl.BlockSpec((1, hid), lambda i: (0, 0)),
            pl.BlockSpec((1, 1), lambda i: (0, 0)),
        ],
        out_specs=pl.BlockSpec((blk, n), lambda i: (i, 0)),
        out_shape=jax.ShapeDtypeStruct((n, n), jnp.float32),
        compiler_params=pltpu.CompilerParams(
            dimension_semantics=("parallel",)),
    )(a, bt, params["lp_w2"].reshape(1, hid), params["lp_b2"].reshape(1, 1))


# speedup vs baseline: 216.1017x; 1.2814x over previous
import jax
import jax.numpy as jnp
from jax.experimental import pallas as pl
from jax.experimental.pallas import tpu as pltpu

_PREC = jax.lax.Precision.HIGHEST


def _embed_kernel(x_ref,
                  wi1_ref, wc1_ref, wo1_ref, bi1_ref, bc1_ref, bo1_ref, wco1_ref,
                  wi2_ref, wc2_ref, wo2_ref, bi2_ref, bc2_ref, bo2_ref, wco2_ref,
                  w1a_ref, b1_ref, w1b_ref,
                  a_ref, bt_ref):
    h = x_ref[:]
    for (wi, wc, wo, bi, bc, bo, wco) in (
        (wi1_ref, wc1_ref, wo1_ref, bi1_ref, bc1_ref, bo1_ref, wco1_ref),
        (wi2_ref, wc2_ref, wo2_ref, bi2_ref, bc2_ref, bo2_ref, wco2_ref),
    ):
        gi = jax.nn.sigmoid(
            jnp.dot(h, wi[:], precision=_PREC, preferred_element_type=jnp.float32)
            + bi[:])
        gt = jnp.tanh(
            jnp.dot(h, wc[:], precision=_PREC, preferred_element_type=jnp.float32)
            + bc[:])
        c = gi * gt
        go = jax.nn.sigmoid(
            jnp.dot(h, wo[:], precision=_PREC, preferred_element_type=jnp.float32)
            + wco[:] * c + bo[:])
        h = go * jnp.tanh(c)

    a_ref[:] = jnp.dot(h, w1a_ref[:], precision=_PREC,
                       preferred_element_type=jnp.float32) + b1_ref[:]
    bt_ref[:] = jnp.dot(h, w1b_ref[:], precision=_PREC,
                        preferred_element_type=jnp.float32).T


def _score_kernel(a_ref, bt_ref, w2_ref, b2_ref, out_ref):
    bt = bt_ref[:]
    w2 = w2_ref[:]
    b2 = b2_ref[:]
    br = a_ref.shape[0]
    bc = bt.shape[1]
    hid = bt.shape[0]
    for i0 in range(0, br, 8):
        a8 = a_ref[i0:i0 + 8, :]
        acc = jnp.broadcast_to(b2, (8, bc))
        for k in range(hid):
            acc = acc + jnp.maximum(a8[:, k:k + 1] + bt[k:k + 1, :], 0.0) * w2[0:1, k:k + 1]
        out_ref[i0:i0 + 8, :] = jax.nn.sigmoid(acc)


def kernel(x, edge_weight, params, edge_index):
    del edge_weight, edge_index
    n = x.shape[0]
    hid = params["lp_w2"].shape[0]

    args = [x]
    for p in params["layers"]:
        args += [
            p["W_i"], p["W_c"], p["W_o"],
            (p["b_i"] + p["conv_i_b"]).astype(jnp.float32),
            (p["b_c"] + p["conv_c_b"]).astype(jnp.float32),
            (p["b_o"] + p["conv_o_b"]).astype(jnp.float32),
            p["w_c_o"],
        ]
    args += [
        params["lp_w1"][:hid],
        params["lp_b1"][None, :],
        params["lp_w1"][hid:],
    ]

    a, bt = pl.pallas_call(
        _embed_kernel,
        out_shape=(jax.ShapeDtypeStruct((n, hid), jnp.float32),
                   jax.ShapeDtypeStruct((hid, n), jnp.float32)),
    )(*args)

    br = 128
    return pl.pallas_call(
        _score_kernel,
        grid=(n // br,),
        in_specs=[
            pl.BlockSpec((br, hid), lambda i: (i, 0)),
            pl.BlockSpec((hid, n), lambda i: (0, 0)),
            pl.BlockSpec((1, hid), lambda i: (0, 0)),
            pl.BlockSpec((1, 1), lambda i: (0, 0)),
        ],
        out_specs=pl.BlockSpec((br, n), lambda i: (i, 0)),
        out_shape=jax.ShapeDtypeStruct((n, n), jnp.float32),
        compiler_params=pltpu.CompilerParams(
            dimension_semantics=("parallel",)),
    )(a, bt, params["lp_w2"].reshape(1, hid), params["lp_b2"].reshape(1, 1))


# speedup vs baseline: 222.7654x; 1.0308x over previous
import jax
import jax.numpy as jnp
from jax.experimental import pallas as pl
from jax.experimental.pallas import tpu as pltpu

_PREC = jax.lax.Precision.HIGHEST


def _embed_kernel(x_ref,
                  w1_ref, b1g_ref, wco1_ref,
                  w2_ref, b2g_ref, wco2_ref,
                  wp_ref, b1_ref,
                  a_ref, bt_ref):
    h = x_ref[:]
    hid = a_ref.shape[1]
    for (w, bg, wco) in ((w1_ref, b1g_ref, wco1_ref),
                         (w2_ref, b2g_ref, wco2_ref)):
        g = jnp.dot(h, w[:], precision=_PREC,
                    preferred_element_type=jnp.float32) + bg[:]
        gi = jax.nn.sigmoid(g[:, :hid])
        gt = jnp.tanh(g[:, hid:2 * hid])
        c = gi * gt
        go = jax.nn.sigmoid(g[:, 2 * hid:] + wco[:] * c)
        h = go * jnp.tanh(c)

    p = jnp.dot(h, wp_ref[:], precision=_PREC,
                preferred_element_type=jnp.float32)
    a_ref[:] = p[:, :hid] + b1_ref[:]
    bt_ref[:] = p[:, hid:].T


def _score_kernel(a_ref, bt_ref, w2_ref, b2_ref, out_ref):
    bt = bt_ref[:]
    w2 = w2_ref[:]
    b2 = b2_ref[:]
    br = a_ref.shape[0]
    bc = bt.shape[1]
    hid = bt.shape[0]
    for i0 in range(0, br, 8):
        a8 = a_ref[i0:i0 + 8, :]
        acc = jnp.broadcast_to(b2, (8, bc))
        for k in range(hid):
            acc = acc + jnp.maximum(a8[:, k:k + 1] + bt[k:k + 1, :], 0.0) * w2[0:1, k:k + 1]
        out_ref[i0:i0 + 8, :] = jax.nn.sigmoid(acc)


def kernel(x, edge_weight, params, edge_index):
    del edge_weight, edge_index
    n = x.shape[0]
    hid = params["lp_w2"].shape[0]

    args = [x]
    for p in params["layers"]:
        args += [
            jnp.concatenate([p["W_i"], p["W_c"], p["W_o"]], axis=1),
            jnp.concatenate([
                (p["b_i"] + p["conv_i_b"][None, :]),
                (p["b_c"] + p["conv_c_b"][None, :]),
                (p["b_o"] + p["conv_o_b"][None, :]),
            ], axis=1).astype(jnp.float32),
            p["w_c_o"],
        ]
    args += [
        jnp.concatenate([params["lp_w1"][:hid], params["lp_w1"][hid:]], axis=1),
        params["lp_b1"][None, :],
    ]

    a, bt = pl.pallas_call(
        _embed_kernel,
        out_shape=(jax.ShapeDtypeStruct((n, hid), jnp.float32),
                   jax.ShapeDtypeStruct((hid, n), jnp.float32)),
    )(*args)

    br = 128
    return pl.pallas_call(
        _score_kernel,
        grid=(n // br,),
        in_specs=[
            pl.BlockSpec((br, hid), lambda i: (i, 0)),
            pl.BlockSpec((hid, n), lambda i: (0, 0)),
            pl.BlockSpec((1, hid), lambda i: (0, 0)),
            pl.BlockSpec((1, 1), lambda i: (0, 0)),
        ],
        out_specs=pl.BlockSpec((br, n), lambda i: (i, 0)),
        out_shape=jax.ShapeDtypeStruct((n, n), jnp.float32),
        compiler_params=pltpu.CompilerParams(
            dimension_semantics=("parallel",)),
    )(a, bt, params["lp_w2"].reshape(1, hid), params["lp_b2"].reshape(1, 1))
